# baseline (speedup 1.0000x reference)
import jax
import jax.numpy as jnp
from jax.experimental import pallas as pl
from jax.experimental.pallas import tpu as pltpu


def _body(t_ref, x_ref, c_ref, w_ref, b_ref, o_ref):
    t = t_ref[0, 0]
    t2 = t * t
    inv = 1.0 / (t + 1.0)
    tinv = t * inv
    centers = c_ref[...]
    w = w_ref[...]
    a1 = (-2.0 * t2) * centers
    c2 = jnp.sum(centers * centers, axis=1, keepdims=True)
    a2 = jnp.concatenate(
        [t2 * c2 + 3e-4 * t2, jnp.full_like(c2, t2)], axis=1)
    wct = jax.lax.dot_general(w, centers, (((1,), (1,)), ((), ())),
                              preferred_element_type=jnp.float32)
    wct_aug = jnp.concatenate(
        [wct, jnp.ones((1, wct.shape[1]), jnp.float32)], axis=0
    ).astype(jnp.bfloat16)

    C, HB, W = x_ref.shape[1], x_ref.shape[2], x_ref.shape[3]
    M = HB * W
    X = x_ref[0].reshape(C, M)
    q2 = jnp.sum(X * X, axis=0, keepdims=True)
    tail = jnp.concatenate([jnp.ones((1, M), jnp.float32), q2], axis=0)
    y = (jax.lax.dot_general(a1, X, (((1,), (0,)), ((), ())),
                             preferred_element_type=jnp.float32)
         + jax.lax.dot_general(a2, tail, (((1,), (0,)), ((), ())),
                               preferred_element_type=jnp.float32))
    e = jax.lax.exp2((y * (-1.4426950408889634)) * jax.lax.rsqrt(y))
    e16 = e.astype(jnp.bfloat16)

    U_aug = jax.lax.dot_general(wct_aug, e16, (((1,), (0,)), ((), ())),
                                preferred_element_type=jnp.float32)
    U = U_aug[:-1]
    sum_e = U_aug[-1:]
    V = jax.lax.dot_general(inv * w, X, (((1,), (0,)), ((), ())),
                            preferred_element_type=jnp.float32)
    out = U * (tinv / sum_e) + V + b_ref[...]
    o_ref[0] = out.reshape(out.shape[0], HB, W)


def kernel(x, weight, bias, cluster_centers, temp):
    B, C, H, W = x.shape
    O = weight.shape[0]
    K = cluster_centers.shape[0]
    HB = 56

    w2 = weight.reshape(O, C)
    bias2 = bias.reshape(O, 1)
    t11 = jnp.asarray(temp, jnp.float32).reshape(1, 1)

    return pl.pallas_call(
        _body,
        grid=(B, H // HB),
        in_specs=[
            pl.BlockSpec((1, 1), lambda b, m: (0, 0)),
            pl.BlockSpec((1, C, HB, W), lambda b, m: (b, 0, m, 0)),
            pl.BlockSpec((K, C), lambda b, m: (0, 0)),
            pl.BlockSpec((O, C), lambda b, m: (0, 0)),
            pl.BlockSpec((O, 1), lambda b, m: (0, 0)),
        ],
        out_specs=pl.BlockSpec((1, O, HB, W), lambda b, m: (b, 0, m, 0)),
        out_shape=jax.ShapeDtypeStruct((B, O, H, W), jnp.float32),
        compiler_params=pltpu.CompilerParams(
            dimension_semantics=("parallel", "parallel"),
        ),
    )(t11, x, cluster_centers, w2, bias2)


# speedup vs baseline: 1.3425x; 1.3425x over previous
import jax
import jax.numpy as jnp
from jax.experimental import pallas as pl
from jax.experimental.pallas import tpu as pltpu


def _body(t_ref, x_ref, c_ref, w_ref, b_ref, o_ref, af_ref, wct_ref):
    t = t_ref[0, 0]
    t2 = t * t
    inv = 1.0 / (t + 1.0)
    tinv = t * inv
    b_idx = pl.program_id(0)
    m_idx = pl.program_id(1)

    K = c_ref.shape[0]
    C = c_ref.shape[1]
    O = w_ref.shape[0]

    @pl.when(jnp.logical_and(b_idx == 0, m_idx == 0))
    def _init():
        centers = c_ref[...]
        w = w_ref[...]
        c2 = jnp.sum(centers * centers, axis=1, keepdims=True)
        top = jnp.concatenate(
            [(-2.0 * t2) * centers, t2 * c2 + 3e-4 * t2,
             jnp.full_like(c2, t2)], axis=1)
        bot = jnp.concatenate(
            [inv * w, jnp.zeros((O, 2), jnp.float32)], axis=1)
        af_ref[...] = jnp.concatenate([top, bot], axis=0)
        wct = jax.lax.dot_general(w, centers, (((1,), (1,)), ((), ())),
                                  preferred_element_type=jnp.float32)
        wct_ref[...] = jnp.concatenate(
            [wct, jnp.ones((1, K), jnp.float32)], axis=0
        ).astype(jnp.bfloat16)

    HB, W = x_ref.shape[2], x_ref.shape[3]
    M = HB * W
    X = x_ref[0].reshape(C, M)
    q2 = jnp.sum(X * X, axis=0, keepdims=True)
    XT = jnp.concatenate([X, jnp.ones((1, M), jnp.float32), q2], axis=0)
    Y = jax.lax.dot_general(af_ref[...], XT, (((1,), (0,)), ((), ())),
                            preferred_element_type=jnp.float32)
    y = Y[:K]
    V = Y[K:]
    e = jax.lax.exp2((y * (-1.4426950408889634)) * jax.lax.rsqrt(y))
    e16 = e.astype(jnp.bfloat16)

    U_aug = jax.lax.dot_general(wct_ref[...], e16, (((1,), (0,)), ((), ())),
                                preferred_element_type=jnp.float32)
    U = U_aug[:-1]
    sum_e = U_aug[-1:]
    out = U * (tinv / sum_e) + V + b_ref[...]
    o_ref[0] = out.reshape(O, HB, W)


def kernel(x, weight, bias, cluster_centers, temp):
    B, C, H, W = x.shape
    O = weight.shape[0]
    K = cluster_centers.shape[0]
    HB = 56

    w2 = weight.reshape(O, C)
    bias2 = bias.reshape(O, 1)
    t11 = jnp.asarray(temp, jnp.float32).reshape(1, 1)

    return pl.pallas_call(
        _body,
        grid=(B, H // HB),
        in_specs=[
            pl.BlockSpec((1, 1), lambda b, m: (0, 0)),
            pl.BlockSpec((1, C, HB, W), lambda b, m: (b, 0, m, 0)),
            pl.BlockSpec((K, C), lambda b, m: (0, 0)),
            pl.BlockSpec((O, C), lambda b, m: (0, 0)),
            pl.BlockSpec((O, 1), lambda b, m: (0, 0)),
        ],
        out_specs=pl.BlockSpec((1, O, HB, W), lambda b, m: (b, 0, m, 0)),
        out_shape=jax.ShapeDtypeStruct((B, O, H, W), jnp.float32),
        scratch_shapes=[
            pltpu.VMEM((K + O, C + 2), jnp.float32),
            pltpu.VMEM((O + 1, K), jnp.bfloat16),
        ],
        compiler_params=pltpu.CompilerParams(
            dimension_semantics=("arbitrary", "arbitrary"),
        ),
    )(t11, x, cluster_centers, w2, bias2)
